# baseline (speedup 1.0000x reference)
import functools

import jax
import jax.numpy as jnp
from jax import lax
from jax.experimental import pallas as pl
from jax.experimental.pallas import tpu as pltpu
from jax.experimental.pallas import tpu_sc as plsc

EMB = 64
SCALE = 8.0
LANES = 16


@functools.lru_cache(maxsize=None)
def _build(l: int, b: int, vocab: int):
    info = plsc.get_sparse_core_info()
    nc, ns = info.num_cores, info.num_subcores
    nw = nc * ns
    gl, gb = 8, 4
    assert gl * gb == nw and l % gl == 0 and b % (gb * 128) == 0
    tl, tb = l // gl, b // gb
    qn = tb // 128
    ebn, ein = EMB // 8, 8
    bgn = 128 // LANES

    mesh = plsc.VectorSubcoreMesh(core_axis_name="c", subcore_axis_name="s")

    @functools.partial(
        pl.kernel,
        mesh=mesh,
        compiler_params=pltpu.CompilerParams(
            use_tc_tiling_on_sc=False, needs_layout_passes=False
        ),
        out_type=jax.ShapeDtypeStruct((l, ebn, b // 128, ein, 128), jnp.float32),
        scratch_types=[
            pltpu.VMEM((tl, tb), jnp.int32),
            pltpu.VMEM((tb, EMB), jnp.float32),
            pltpu.VMEM((tb, EMB), jnp.float32),
            pltpu.VMEM((ebn, qn, ein, 128), jnp.float32),
            pltpu.VMEM((ebn, qn, ein, 128), jnp.float32),
            pltpu.SemaphoreType.DMA,
            pltpu.SemaphoreType.DMA,
            pltpu.SemaphoreType.DMA,
            pltpu.SemaphoreType.DMA,
        ],
    )
    def gather_transpose(
        table_hbm, idx_hbm, out_hbm,
        idx_v, in0, in1, t0, t1, g0, g1, s0, s1,
    ):
        ins = (in0, in1)
        ts = (t0, t1)
        gsems = (g0, g1)
        ssems = (s0, s1)
        wid = lax.axis_index("s") * nc + lax.axis_index("c")
        l0 = (wid // gb) * tl
        bb0 = (wid % gb) * qn
        pltpu.sync_copy(
            idx_hbm.at[pl.ds(l0, tl), pl.ds(bb0 * 128, tb)], idx_v
        )
        lanes = lax.iota(jnp.int32, LANES)

        def gather_start(c, bf):
            pltpu.async_copy(table_hbm.at[idx_v.at[c]], ins[bf], gsems[bf])

        def out_copy(c, bf):
            return pltpu.make_async_copy(
                ts[bf],
                out_hbm.at[l0 + c, slice(None), pl.ds(bb0, qn)],
                ssems[bf],
            )

        gather_start(0, 0)
        gather_start(1, 1)

        for c in range(tl):
            bf = c % 2
            pltpu.make_async_copy(
                table_hbm.at[idx_v.at[c]], ins[bf], gsems[bf]
            ).wait()
            if c >= 2:
                out_copy(c - 2, bf).wait()

            def trans_body(e, _, bf=bf):
                eb = e // ein
                ei = e % ein
                for q in range(qn):
                    for bg in range(bgn):
                        rows = lanes + (q * 128 + bg * LANES)
                        cols = jnp.full((LANES,), e, jnp.int32)
                        v = plsc.load_gather(ins[bf], [rows, cols])
                        ts[bf][eb, q, ei, pl.ds(bg * LANES, LANES)] = v * SCALE
                return 0

            lax.fori_loop(0, EMB, trans_body, 0)

            if c + 2 < tl:
                gather_start(c + 2, bf)
            out_copy(c, bf).start()

        out_copy(tl - 2, (tl - 2) % 2).wait()
        out_copy(tl - 1, (tl - 1) % 2).wait()

    return gather_transpose


def kernel(src, SenEmbedding_dict, embedding_weight):
    l, b = src.shape
    vocab, emb = embedding_weight.shape
    fn = _build(l, b, vocab)
    y = fn(embedding_weight, src.astype(jnp.int32))
    return y.transpose(0, 2, 4, 1, 3).reshape(l, b, emb)


# speedup vs baseline: 1.6217x; 1.6217x over previous
import functools

import jax
import jax.numpy as jnp
from jax import lax
from jax.experimental import pallas as pl
from jax.experimental.pallas import tpu as pltpu
from jax.experimental.pallas import tpu_sc as plsc

EMB = 64
SCALE = 8.0


@functools.lru_cache(maxsize=None)
def _build_gather(l: int, b: int, vocab: int):
    info = plsc.get_sparse_core_info()
    nc, ns = info.num_cores, info.num_subcores
    nw = nc * ns
    gl, gb = 8, 4
    assert gl * gb == nw and l % gl == 0 and b % gb == 0
    tl, tb = l // gl, b // gb

    mesh = plsc.VectorSubcoreMesh(core_axis_name="c", subcore_axis_name="s")

    @functools.partial(
        pl.kernel,
        mesh=mesh,
        compiler_params=pltpu.CompilerParams(
            use_tc_tiling_on_sc=False, needs_layout_passes=False
        ),
        out_type=jax.ShapeDtypeStruct((l * b // 2, 2 * EMB), jnp.float32),
        scratch_types=[
            pltpu.VMEM((tl, tb), jnp.int32),
            pltpu.VMEM((tb, EMB), jnp.float32),
            pltpu.VMEM((tb, EMB), jnp.float32),
            pltpu.SemaphoreType.DMA,
            pltpu.SemaphoreType.DMA,
            pltpu.SemaphoreType.DMA,
            pltpu.SemaphoreType.DMA,
        ],
    )
    def gather_k(table_hbm, idx_hbm, out_hbm, idx_v, in0, in1, g0, g1, s0, s1):
        ins = (in0, in1)
        gsems = (g0, g1)
        ssems = (s0, s1)
        wid = lax.axis_index("s") * nc + lax.axis_index("c")
        l0 = (wid // gb) * tl
        b0 = (wid % gb) * tb
        pltpu.sync_copy(idx_hbm.at[pl.ds(l0, tl), pl.ds(b0, tb)], idx_v)

        def gather_start(c, bf):
            pltpu.async_copy(table_hbm.at[idx_v.at[c]], ins[bf], gsems[bf])

        hb = b // 2
        p0 = b0 // hb
        r0 = b0 - p0 * hb

        def out_copy(c, bf):
            return pltpu.make_async_copy(
                ins[bf],
                out_hbm.at[
                    pl.ds((l0 + c) * hb + r0, tb), pl.ds(p0 * EMB, EMB)
                ],
                ssems[bf],
            )

        gather_start(0, 0)
        gather_start(1, 1)
        for c in range(tl):
            bf = c % 2
            pltpu.make_async_copy(
                table_hbm.at[idx_v.at[c]], ins[bf], gsems[bf]
            ).wait()
            out_copy(c, bf).start()
            if c + 2 < tl:
                out_copy(c, bf).wait()
                gather_start(c + 2, bf)
        out_copy(tl - 2, (tl - 2) % 2).wait()
        out_copy(tl - 1, (tl - 1) % 2).wait()

    return gather_k


@functools.lru_cache(maxsize=None)
def _build_transpose(l: int, b: int):
    bbn = b // 128
    ebn, ein = EMB // 8, 8
    hb = b // 2

    def trans_k(x_ref, y_ref):
        x = x_ref[...]
        xt = jnp.concatenate([x[:, :EMB].T, x[:, EMB:].T], axis=1)
        xt = xt * SCALE
        y = xt.reshape(ebn, ein, bbn, 128).transpose(0, 2, 1, 3)
        y_ref[...] = y.reshape(1, ebn, bbn, ein, 128)

    return pl.pallas_call(
        trans_k,
        grid=(l,),
        in_specs=[pl.BlockSpec((hb, 2 * EMB), lambda i: (i, 0))],
        out_specs=pl.BlockSpec(
            (1, ebn, bbn, ein, 128), lambda i: (i, 0, 0, 0, 0)
        ),
        out_shape=jax.ShapeDtypeStruct((l, ebn, bbn, ein, 128), jnp.float32),
    )


def kernel(src, SenEmbedding_dict, embedding_weight):
    l, b = src.shape
    vocab, emb = embedding_weight.shape
    x = _build_gather(l, b, vocab)(embedding_weight, src.astype(jnp.int32))
    y = _build_transpose(l, b)(x)
    return y.transpose(0, 2, 4, 1, 3).reshape(l, b, emb)


# speedup vs baseline: 1.6338x; 1.0075x over previous
import functools

import jax
import jax.numpy as jnp
from jax import lax
from jax.experimental import pallas as pl
from jax.experimental.pallas import tpu as pltpu
from jax.experimental.pallas import tpu_sc as plsc

EMB = 64
SCALE = 8.0


@functools.lru_cache(maxsize=None)
def _build_gather(l: int, b: int, vocab: int):
    info = plsc.get_sparse_core_info()
    nc, ns = info.num_cores, info.num_subcores
    nw = nc * ns
    gl, gb = 8, 4
    assert gl * gb == nw and l % gl == 0 and b % gb == 0
    tl, tb = l // gl, b // gb

    mesh = plsc.VectorSubcoreMesh(core_axis_name="c", subcore_axis_name="s")

    @functools.partial(
        pl.kernel,
        mesh=mesh,
        compiler_params=pltpu.CompilerParams(
            use_tc_tiling_on_sc=False, needs_layout_passes=False
        ),
        out_type=jax.ShapeDtypeStruct((l * b // 2, 2 * EMB), jnp.float32),
        scratch_types=[
            pltpu.VMEM((tl, tb), jnp.int32),
            pltpu.VMEM((tb, EMB), jnp.float32),
            pltpu.VMEM((tb, EMB), jnp.float32),
            pltpu.SemaphoreType.DMA,
            pltpu.SemaphoreType.DMA,
            pltpu.SemaphoreType.DMA,
            pltpu.SemaphoreType.DMA,
        ],
    )
    def gather_k(table_hbm, idx_hbm, out_hbm, idx_v, in0, in1, g0, g1, s0, s1):
        ins = (in0, in1)
        gsems = (g0, g1)
        ssems = (s0, s1)
        wid = lax.axis_index("s") * nc + lax.axis_index("c")
        l0 = (wid // gb) * tl
        b0 = (wid % gb) * tb
        pltpu.sync_copy(idx_hbm.at[pl.ds(l0, tl), pl.ds(b0, tb)], idx_v)

        def gather_start(c, bf):
            pltpu.async_copy(table_hbm.at[idx_v.at[c]], ins[bf], gsems[bf])

        hb = b // 2
        p0 = b0 // hb
        r0 = b0 - p0 * hb

        def out_copy(c, bf):
            return pltpu.make_async_copy(
                ins[bf],
                out_hbm.at[
                    pl.ds((l0 + c) * hb + r0, tb), pl.ds(p0 * EMB, EMB)
                ],
                ssems[bf],
            )

        gather_start(0, 0)
        gather_start(1, 1)
        for c in range(tl):
            bf = c % 2
            pltpu.make_async_copy(
                table_hbm.at[idx_v.at[c]], ins[bf], gsems[bf]
            ).wait()
            out_copy(c, bf).start()
            if c + 2 < tl:
                out_copy(c, bf).wait()
                gather_start(c + 2, bf)
        out_copy(tl - 2, (tl - 2) % 2).wait()
        out_copy(tl - 1, (tl - 1) % 2).wait()

    return gather_k


@functools.lru_cache(maxsize=None)
def _build_transpose(l: int, b: int):
    bbn = b // 128
    ebn, ein = EMB // 8, 8
    hb = b // 2

    dn = (((1,), (1,)), ((), ()))

    def trans_k(x_ref, y_ref):
        x = x_ref[...]
        eye8 = jnp.eye(EMB, dtype=jnp.float32) * SCALE
        at = lax.dot_general(eye8, x[:, :EMB], dn,
                             preferred_element_type=jnp.float32)
        bt = lax.dot_general(eye8, x[:, EMB:], dn,
                             preferred_element_type=jnp.float32)
        xt = jnp.concatenate([at, bt], axis=1)
        y = xt.reshape(ebn, ein, bbn, 128).transpose(0, 2, 1, 3)
        y_ref[...] = y.reshape(1, ebn, bbn, ein, 128)

    return pl.pallas_call(
        trans_k,
        grid=(l,),
        in_specs=[pl.BlockSpec((hb, 2 * EMB), lambda i: (i, 0))],
        out_specs=pl.BlockSpec(
            (1, ebn, bbn, ein, 128), lambda i: (i, 0, 0, 0, 0)
        ),
        out_shape=jax.ShapeDtypeStruct((l, ebn, bbn, ein, 128), jnp.float32),
    )


def kernel(src, SenEmbedding_dict, embedding_weight):
    l, b = src.shape
    vocab, emb = embedding_weight.shape
    x = _build_gather(l, b, vocab)(embedding_weight, src.astype(jnp.int32))
    y = _build_transpose(l, b)(x)
    return y.transpose(0, 2, 4, 1, 3).reshape(l, b, emb)


# speedup vs baseline: 2.5586x; 1.5660x over previous
import functools

import jax
import jax.numpy as jnp
from jax import lax
from jax.experimental import pallas as pl
from jax.experimental.pallas import tpu as pltpu
from jax.experimental.pallas import tpu_sc as plsc

EMB = 64
SCALE = 8.0


@functools.lru_cache(maxsize=None)
def _build_gather(l: int, b: int, vocab: int):
    info = plsc.get_sparse_core_info()
    nc, ns = info.num_cores, info.num_subcores
    nw = nc * ns
    gl, gb = 8, 4
    assert gl * gb == nw and l % gl == 0 and b % gb == 0
    tl, tb = l // gl, b // gb

    mesh = plsc.VectorSubcoreMesh(core_axis_name="c", subcore_axis_name="s")

    @functools.partial(
        pl.kernel,
        mesh=mesh,
        compiler_params=pltpu.CompilerParams(
            use_tc_tiling_on_sc=False, needs_layout_passes=False
        ),
        out_type=jax.ShapeDtypeStruct((l * b // 2, 2 * EMB), jnp.float32),
        scratch_types=[
            pltpu.VMEM((tl, tb), jnp.int32),
            pltpu.VMEM((tb, EMB), jnp.float32),
            pltpu.VMEM((tb, EMB), jnp.float32),
            pltpu.SemaphoreType.DMA,
            pltpu.SemaphoreType.DMA,
            pltpu.SemaphoreType.DMA,
            pltpu.SemaphoreType.DMA,
        ],
    )
    def gather_k(table_hbm, idx_hbm, out_hbm, idx_v, in0, in1, g0, g1, s0, s1):
        ins = (in0, in1)
        gsems = (g0, g1)
        ssems = (s0, s1)
        wid = lax.axis_index("s") * nc + lax.axis_index("c")
        l0 = (wid // gb) * tl
        b0 = (wid % gb) * tb
        pltpu.sync_copy(idx_hbm.at[pl.ds(l0, tl), pl.ds(b0, tb)], idx_v)

        def gather_start(c, bf):
            pltpu.async_copy(table_hbm.at[idx_v.at[c]], ins[bf], gsems[bf])

        hb = b // 2
        p0 = b0 // hb
        r0 = b0 - p0 * hb

        def out_copy(c, bf):
            return pltpu.make_async_copy(
                ins[bf],
                out_hbm.at[
                    pl.ds((l0 + c) * hb + r0, tb), pl.ds(p0 * EMB, EMB)
                ],
                ssems[bf],
            )

        gather_start(0, 0)
        gather_start(1, 1)
        for c in range(tl):
            bf = c % 2
            pltpu.make_async_copy(
                table_hbm.at[idx_v.at[c]], ins[bf], gsems[bf]
            ).wait()
            out_copy(c, bf).start()
            if c + 2 < tl:
                out_copy(c, bf).wait()
                gather_start(c + 2, bf)
        out_copy(tl - 2, (tl - 2) % 2).wait()
        out_copy(tl - 1, (tl - 1) % 2).wait()

    return gather_k


@functools.lru_cache(maxsize=None)
def _build_transpose(l: int, b: int):
    bbn = b // 128
    ebn, ein = EMB // 8, 8
    hb = b // 2

    dn = (((1,), (1,)), ((), ()))
    lc = 8

    def trans_k(x_ref, y_ref):
        eye8 = jnp.eye(EMB, dtype=jnp.float32) * SCALE
        for j in range(lc):
            x = x_ref[pl.ds(j * hb, hb), :]
            at = lax.dot_general(eye8, x[:, :EMB], dn,
                                 preferred_element_type=jnp.float32)
            bt = lax.dot_general(eye8, x[:, EMB:], dn,
                                 preferred_element_type=jnp.float32)
            xt = jnp.concatenate([at, bt], axis=1)
            y = xt.reshape(ebn, ein, bbn, 128).transpose(0, 2, 1, 3)
            y_ref[j] = y.reshape(ebn, bbn, ein, 128)

    return pl.pallas_call(
        trans_k,
        grid=(l // lc,),
        in_specs=[pl.BlockSpec((lc * hb, 2 * EMB), lambda i: (i, 0))],
        out_specs=pl.BlockSpec(
            (lc, ebn, bbn, ein, 128), lambda i: (i, 0, 0, 0, 0)
        ),
        out_shape=jax.ShapeDtypeStruct((l, ebn, bbn, ein, 128), jnp.float32),
    )


def kernel(src, SenEmbedding_dict, embedding_weight):
    l, b = src.shape
    vocab, emb = embedding_weight.shape
    x = _build_gather(l, b, vocab)(embedding_weight, src.astype(jnp.int32))
    y = _build_transpose(l, b)(x)
    return y.transpose(0, 2, 4, 1, 3).reshape(l, b, emb)
